# Initial kernel scaffold; baseline (speedup 1.0000x reference)
import jax
import jax.numpy as jnp
from jax.experimental import pallas as pl


def kernel(anchor_bboxes, n_level_bboxes, gt_bboxes, gt_labels, mask_gt, pred_bboxes):
    raise NotImplementedError("write your pallas kernel here")


# speedup vs baseline: 12.8277x; 12.8277x over previous
import functools

import jax
import jax.numpy as jnp
from jax.experimental import pallas as pl

TOPK = 9
NUM_CLASSES = 80
BG_INDEX = 80
EPS = 1e-9


def _atss_body(levels, n, m, an_ref, gt_ref, lab_ref, pred_ref,
               lab_out, box_out, sc_out, fg_out):
    an = an_ref[...]
    gt = gt_ref[0]
    lab = lab_ref[0]
    pred = pred_ref[0]

    ax1 = an[0:1, :]
    ay1 = an[1:2, :]
    ax2 = an[2:3, :]
    ay2 = an[3:4, :]
    gx1 = gt[:, 0:1]
    gy1 = gt[:, 1:2]
    gx2 = gt[:, 2:3]
    gy2 = gt[:, 3:4]

    w = jnp.maximum(jnp.minimum(gx2, ax2) - jnp.maximum(gx1, ax1), 0.0)
    h = jnp.maximum(jnp.minimum(gy2, ay2) - jnp.maximum(gy1, ay1), 0.0)
    inter = w * h
    a_gt = jnp.maximum(gx2 - gx1, 0.0) * jnp.maximum(gy2 - gy1, 0.0)
    a_an = jnp.maximum(ax2 - ax1, 0.0) * jnp.maximum(ay2 - ay1, 0.0)
    overlaps = inter / (a_gt + a_an - inter + EPS)

    gcx = (gx1 + gx2) / 2.0
    gcy = (gy1 + gy2) / 2.0
    acx = (ax1 + ax2) / 2.0
    acy = (ay1 + ay2) / 2.0
    dx = gcx - acx
    dy = gcy - acy
    dist = jnp.sqrt(dx ** 2 + dy ** 2 + 1e-12)

    iota_m = jax.lax.broadcasted_iota(jnp.int32, (1, m), 1)
    iota_n = jax.lax.broadcasted_iota(jnp.int32, (n, 1), 0)

    cand = jnp.zeros((n, m), jnp.bool_)
    start = 0
    for nb in levels:
        lvl_mask = (iota_m >= start) & (iota_m < start + nb)
        dw = jnp.where(lvl_mask, dist, jnp.inf)
        for _ in range(min(TOPK, nb)):
            dmin = jnp.min(dw, axis=1, keepdims=True)
            imin = jnp.min(jnp.where(dw == dmin, iota_m, m), axis=1,
                           keepdims=True)
            sel = iota_m == imin
            cand = cand | sel
            dw = jnp.where(sel, jnp.inf, dw)
        start += nb

    k_total = sum(min(TOPK, nb) for nb in levels)
    cand_o = jnp.where(cand, overlaps, 0.0)
    mean = jnp.sum(cand_o, axis=1, keepdims=True) / k_total
    dev = jnp.where(cand, overlaps - mean, 0.0)
    var = jnp.sum(dev * dev, axis=1, keepdims=True) / (k_total - 1)
    thr = mean + jnp.sqrt(var)
    is_pos = cand & (overlaps > thr)

    in_gt = (jnp.minimum(jnp.minimum(acx - gx1, acy - gy1),
                         jnp.minimum(gx2 - acx, gy2 - acy)) > EPS)
    mask_pos = (is_pos & in_gt).astype(jnp.float32)

    fg = jnp.sum(mask_pos, axis=0, keepdims=True)
    omax = jnp.max(overlaps, axis=0, keepdims=True)
    nmax = jnp.min(jnp.where(overlaps == omax, iota_n, n), axis=0,
                   keepdims=True)
    is_max = (iota_n == nmax).astype(jnp.float32)
    mask_pos = jnp.where(fg > 1.0, is_max, mask_pos)
    fg = jnp.sum(mask_pos, axis=0, keepdims=True)
    first = jnp.min(jnp.where(mask_pos > 0.5, iota_n, n), axis=0,
                    keepdims=True)
    tgt = jnp.where(fg > 0.0, first, 0)

    is_sel = iota_n == tgt
    lab_sel = jnp.sum(jnp.where(is_sel, lab, 0), axis=0, keepdims=True)
    labels = jnp.where(fg > 0.0, lab_sel, BG_INDEX)
    lab_out[0] = labels
    fg_out[0] = (fg > 0.0).astype(jnp.int32)

    sel_f = is_sel.astype(jnp.float32)
    rows = [jnp.sum(sel_f * gt[:, c:c + 1], axis=0, keepdims=True)
            for c in range(4)]
    box_out[0] = jnp.concatenate(rows, axis=0)

    px1 = pred[0:1, :]
    py1 = pred[1:2, :]
    px2 = pred[2:3, :]
    py2 = pred[3:4, :]
    wp = jnp.maximum(jnp.minimum(gx2, px2) - jnp.maximum(gx1, px1), 0.0)
    hp = jnp.maximum(jnp.minimum(gy2, py2) - jnp.maximum(gy1, py1), 0.0)
    interp = wp * hp
    a_pr = jnp.maximum(px2 - px1, 0.0) * jnp.maximum(py2 - py1, 0.0)
    iou_pred = interp / (a_gt + a_pr - interp + EPS)
    ioum = jnp.max(iou_pred * mask_pos, axis=0, keepdims=True)
    cls_iota = jax.lax.broadcasted_iota(jnp.int32, (NUM_CLASSES, m), 0)
    sc_out[0] = jnp.where(cls_iota == labels, ioum, 0.0)


def _run(levels, anchors_t, gt_bboxes, labels_i, pred_t, interpret=False):
    b, n = gt_bboxes.shape[0], gt_bboxes.shape[1]
    m = anchors_t.shape[1]
    body = functools.partial(_atss_body, levels, n, m)
    out_shapes = (
        jax.ShapeDtypeStruct((b, 1, m), jnp.int32),
        jax.ShapeDtypeStruct((b, 4, m), jnp.float32),
        jax.ShapeDtypeStruct((b, NUM_CLASSES, m), jnp.float32),
        jax.ShapeDtypeStruct((b, 1, m), jnp.int32),
    )
    return pl.pallas_call(
        body,
        grid=(b,),
        in_specs=[
            pl.BlockSpec((4, m), lambda i: (0, 0)),
            pl.BlockSpec((1, n, 4), lambda i: (i, 0, 0)),
            pl.BlockSpec((1, n, 1), lambda i: (i, 0, 0)),
            pl.BlockSpec((1, 4, m), lambda i: (i, 0, 0)),
        ],
        out_specs=(
            pl.BlockSpec((1, 1, m), lambda i: (i, 0, 0)),
            pl.BlockSpec((1, 4, m), lambda i: (i, 0, 0)),
            pl.BlockSpec((1, NUM_CLASSES, m), lambda i: (i, 0, 0)),
            pl.BlockSpec((1, 1, m), lambda i: (i, 0, 0)),
        ),
        out_shape=out_shapes,
        interpret=interpret,
    )(anchors_t, gt_bboxes, labels_i, pred_t)


def kernel(anchor_bboxes, n_level_bboxes, gt_bboxes, gt_labels, mask_gt,
           pred_bboxes, interpret=False):
    n_lv = len(n_level_bboxes)
    m = anchor_bboxes.shape[0]
    base = m // sum(4 ** j for j in range(n_lv))
    levels = tuple(base * (4 ** (n_lv - 1 - i)) for i in range(n_lv))

    anchors_t = anchor_bboxes.T.astype(jnp.float32)
    pred_t = jnp.transpose(pred_bboxes, (0, 2, 1)).astype(jnp.float32)
    labels_i = gt_labels.astype(jnp.int32)
    gt_f = gt_bboxes.astype(jnp.float32)

    labels, box_t, sc_t, fg = _run(levels, anchors_t, gt_f, labels_i,
                                   pred_t, interpret=interpret)
    target_bboxes = jnp.transpose(box_t, (0, 2, 1))
    target_scores = jnp.transpose(sc_t, (0, 2, 1))
    return labels[:, 0, :], target_bboxes, target_scores, fg[:, 0, :] > 0


# speedup vs baseline: 25.9137x; 2.0201x over previous
import functools

import jax
import jax.numpy as jnp
from jax.experimental import pallas as pl
from jax.experimental.pallas import tpu as pltpu

TOPK = 9
NUM_CLASSES = 80
BG_INDEX = 80
EPS = 1e-9


def _atss_body(levels, n, m, an_ref, gt_ref, lab_ref, pred_ref,
               lab_out, box_out, sc_out, fg_out):
    an = an_ref[...]
    gt = gt_ref[0]
    lab = lab_ref[0]
    pred = pred_ref[0]

    ax1 = an[0:1, :]
    ay1 = an[1:2, :]
    ax2 = an[2:3, :]
    ay2 = an[3:4, :]
    gx1 = gt[:, 0:1]
    gy1 = gt[:, 1:2]
    gx2 = gt[:, 2:3]
    gy2 = gt[:, 3:4]

    w = jnp.maximum(jnp.minimum(gx2, ax2) - jnp.maximum(gx1, ax1), 0.0)
    h = jnp.maximum(jnp.minimum(gy2, ay2) - jnp.maximum(gy1, ay1), 0.0)
    inter = w * h
    a_gt = jnp.maximum(gx2 - gx1, 0.0) * jnp.maximum(gy2 - gy1, 0.0)
    a_an = jnp.maximum(ax2 - ax1, 0.0) * jnp.maximum(ay2 - ay1, 0.0)
    overlaps = inter / (a_gt + a_an - inter + EPS)

    gcx = (gx1 + gx2) / 2.0
    gcy = (gy1 + gy2) / 2.0
    acx = (ax1 + ax2) / 2.0
    acy = (ay1 + ay2) / 2.0
    dx = gcx - acx
    dy = gcy - acy
    dist = jnp.sqrt(dx ** 2 + dy ** 2 + 1e-12)

    iota_n = jax.lax.broadcasted_iota(jnp.int32, (n, 1), 0)

    cand_parts = []
    start = 0
    for nb in levels:
        iota_l = jax.lax.broadcasted_iota(jnp.int32, (1, nb), 1)
        dw = dist[:, start:start + nb]
        cl = jnp.zeros((n, nb), jnp.float32)
        for _ in range(min(TOPK, nb)):
            dmin = jnp.min(dw, axis=1, keepdims=True)
            imin = jnp.min(jnp.where(dw == dmin, iota_l, nb), axis=1,
                           keepdims=True)
            sel = iota_l == imin
            cl = jnp.where(sel, 1.0, cl)
            dw = jnp.where(sel, jnp.inf, dw)
        cand_parts.append(cl)
        start += nb
    cand = jnp.concatenate(cand_parts, axis=1) > 0.5

    k_total = sum(min(TOPK, nb) for nb in levels)
    cand_o = jnp.where(cand, overlaps, 0.0)
    mean = jnp.sum(cand_o, axis=1, keepdims=True) / k_total
    dev = jnp.where(cand, overlaps - mean, 0.0)
    var = jnp.sum(dev * dev, axis=1, keepdims=True) / (k_total - 1)
    thr = mean + jnp.sqrt(var)
    is_pos = cand & (overlaps > thr)

    in_gt = (jnp.minimum(jnp.minimum(acx - gx1, acy - gy1),
                         jnp.minimum(gx2 - acx, gy2 - acy)) > EPS)
    mask_pos = (is_pos & in_gt).astype(jnp.float32)

    fg = jnp.sum(mask_pos, axis=0, keepdims=True)
    omax = jnp.max(overlaps, axis=0, keepdims=True)
    nmax = jnp.min(jnp.where(overlaps == omax, iota_n, n), axis=0,
                   keepdims=True)
    is_max = (iota_n == nmax).astype(jnp.float32)
    mask_pos = jnp.where(fg > 1.0, is_max, mask_pos)
    fg = jnp.sum(mask_pos, axis=0, keepdims=True)
    first = jnp.min(jnp.where(mask_pos > 0.5, iota_n, n), axis=0,
                    keepdims=True)
    tgt = jnp.where(fg > 0.0, first, 0)

    is_sel = iota_n == tgt
    lab_sel = jnp.sum(jnp.where(is_sel, lab, 0), axis=0, keepdims=True)
    labels = jnp.where(fg > 0.0, lab_sel, BG_INDEX)
    lab_out[0] = labels
    fg_out[0] = (fg > 0.0).astype(jnp.int32)

    sel_f = is_sel.astype(jnp.float32)
    rows = [jnp.sum(sel_f * gt[:, c:c + 1], axis=0, keepdims=True)
            for c in range(4)]
    box_out[0] = jnp.concatenate(rows, axis=0)

    px1 = pred[0:1, :]
    py1 = pred[1:2, :]
    px2 = pred[2:3, :]
    py2 = pred[3:4, :]
    wp = jnp.maximum(jnp.minimum(gx2, px2) - jnp.maximum(gx1, px1), 0.0)
    hp = jnp.maximum(jnp.minimum(gy2, py2) - jnp.maximum(gy1, py1), 0.0)
    interp = wp * hp
    a_pr = jnp.maximum(px2 - px1, 0.0) * jnp.maximum(py2 - py1, 0.0)
    iou_pred = interp / (a_gt + a_pr - interp + EPS)
    ioum = jnp.max(iou_pred * mask_pos, axis=0, keepdims=True)
    cls_iota = jax.lax.broadcasted_iota(jnp.int32, (NUM_CLASSES, m), 0)
    sc_out[0] = jnp.where(cls_iota == labels, ioum, 0.0)


def _run(levels, anchors_t, gt_bboxes, labels_i, pred_t, interpret=False):
    b, n = gt_bboxes.shape[0], gt_bboxes.shape[1]
    m = anchors_t.shape[1]
    body = functools.partial(_atss_body, levels, n, m)
    out_shapes = (
        jax.ShapeDtypeStruct((b, 1, m), jnp.int32),
        jax.ShapeDtypeStruct((b, 4, m), jnp.float32),
        jax.ShapeDtypeStruct((b, NUM_CLASSES, m), jnp.float32),
        jax.ShapeDtypeStruct((b, 1, m), jnp.int32),
    )
    return pl.pallas_call(
        body,
        grid=(b,),
        in_specs=[
            pl.BlockSpec((4, m), lambda i: (0, 0)),
            pl.BlockSpec((1, n, 4), lambda i: (i, 0, 0)),
            pl.BlockSpec((1, n, 1), lambda i: (i, 0, 0)),
            pl.BlockSpec((1, 4, m), lambda i: (i, 0, 0)),
        ],
        out_specs=(
            pl.BlockSpec((1, 1, m), lambda i: (i, 0, 0)),
            pl.BlockSpec((1, 4, m), lambda i: (i, 0, 0)),
            pl.BlockSpec((1, NUM_CLASSES, m), lambda i: (i, 0, 0)),
            pl.BlockSpec((1, 1, m), lambda i: (i, 0, 0)),
        ),
        out_shape=out_shapes,
        compiler_params=pltpu.CompilerParams(
            dimension_semantics=("parallel",)),
        interpret=interpret,
    )(anchors_t, gt_bboxes, labels_i, pred_t)


def kernel(anchor_bboxes, n_level_bboxes, gt_bboxes, gt_labels, mask_gt,
           pred_bboxes, interpret=False):
    n_lv = len(n_level_bboxes)
    m = anchor_bboxes.shape[0]
    base = m // sum(4 ** j for j in range(n_lv))
    levels = tuple(base * (4 ** (n_lv - 1 - i)) for i in range(n_lv))

    anchors_t = anchor_bboxes.T.astype(jnp.float32)
    pred_t = jnp.transpose(pred_bboxes, (0, 2, 1)).astype(jnp.float32)
    labels_i = gt_labels.astype(jnp.int32)
    gt_f = gt_bboxes.astype(jnp.float32)

    labels, box_t, sc_t, fg = _run(levels, anchors_t, gt_f, labels_i,
                                   pred_t, interpret=interpret)
    target_bboxes = jnp.transpose(box_t, (0, 2, 1))
    target_scores = jnp.transpose(sc_t, (0, 2, 1))
    return labels[:, 0, :], target_bboxes, target_scores, fg[:, 0, :] > 0


# speedup vs baseline: 30.7068x; 1.1850x over previous
import functools

import jax
import jax.numpy as jnp
from jax.experimental import pallas as pl
from jax.experimental.pallas import tpu as pltpu

TOPK = 9
NUM_CLASSES = 80
BG_INDEX = 80
EPS = 1e-9


def _atss_body(levels, n, m, an_ref, gt_ref, lab_ref, pred_ref,
               lab_out, box_out, sc_out, fg_out):
    an = an_ref[...]
    gt = gt_ref[0]
    lab = lab_ref[0]
    pred = pred_ref[0]

    ax1 = an[0:1, :]
    ay1 = an[1:2, :]
    ax2 = an[2:3, :]
    ay2 = an[3:4, :]
    gx1 = gt[:, 0:1]
    gy1 = gt[:, 1:2]
    gx2 = gt[:, 2:3]
    gy2 = gt[:, 3:4]

    w = jnp.maximum(jnp.minimum(gx2, ax2) - jnp.maximum(gx1, ax1), 0.0)
    h = jnp.maximum(jnp.minimum(gy2, ay2) - jnp.maximum(gy1, ay1), 0.0)
    inter = w * h
    a_gt = jnp.maximum(gx2 - gx1, 0.0) * jnp.maximum(gy2 - gy1, 0.0)
    a_an = jnp.maximum(ax2 - ax1, 0.0) * jnp.maximum(ay2 - ay1, 0.0)
    overlaps = inter / (a_gt + a_an - inter + EPS)

    gcx = (gx1 + gx2) / 2.0
    gcy = (gy1 + gy2) / 2.0
    acx = (ax1 + ax2) / 2.0
    acy = (ay1 + ay2) / 2.0
    dx = gcx - acx
    dy = gcy - acy
    dist = jnp.sqrt(dx ** 2 + dy ** 2 + 1e-12)

    iota_n = jax.lax.broadcasted_iota(jnp.int32, (n, 1), 0)

    cand_parts = []
    start = 0
    for nb in levels:
        iota_l = jax.lax.broadcasted_iota(jnp.int32, (1, nb), 1)
        dw = dist[:, start:start + nb]
        for _ in range(min(TOPK, nb)):
            dmin = jnp.min(dw, axis=1, keepdims=True)
            imin = jnp.min(jnp.where(dw == dmin, iota_l, nb), axis=1,
                           keepdims=True)
            dw = jnp.where(iota_l == imin, jnp.inf, dw)
        cand_parts.append(jnp.isinf(dw).astype(jnp.float32))
        start += nb
    cand = jnp.concatenate(cand_parts, axis=1) > 0.5

    k_total = sum(min(TOPK, nb) for nb in levels)
    cand_o = jnp.where(cand, overlaps, 0.0)
    mean = jnp.sum(cand_o, axis=1, keepdims=True) / k_total
    dev = jnp.where(cand, overlaps - mean, 0.0)
    var = jnp.sum(dev * dev, axis=1, keepdims=True) / (k_total - 1)
    thr = mean + jnp.sqrt(var)
    is_pos = cand & (overlaps > thr)

    in_gt = (jnp.minimum(jnp.minimum(acx - gx1, acy - gy1),
                         jnp.minimum(gx2 - acx, gy2 - acy)) > EPS)
    mask_pos = (is_pos & in_gt).astype(jnp.float32)

    fg = jnp.sum(mask_pos, axis=0, keepdims=True)
    omax = jnp.max(overlaps, axis=0, keepdims=True)
    nmax = jnp.min(jnp.where(overlaps == omax, iota_n, n), axis=0,
                   keepdims=True)
    first = jnp.min(jnp.where(mask_pos > 0.5, iota_n, n), axis=0,
                    keepdims=True)
    tgt = jnp.where(fg > 1.0, nmax,
                    jnp.where(fg > 0.0, first, 0))

    is_sel = iota_n == tgt
    lab_sel = jnp.sum(jnp.where(is_sel, lab, 0), axis=0, keepdims=True)
    labels = jnp.where(fg > 0.0, lab_sel, BG_INDEX)
    lab_out[0] = labels
    fg_out[0] = (fg > 0.0).astype(jnp.int32)

    sel_f = is_sel.astype(jnp.float32)
    rows = [jnp.sum(sel_f * gt[:, c:c + 1], axis=0, keepdims=True)
            for c in range(4)]
    box_out[0] = jnp.concatenate(rows, axis=0)

    tbx1, tby1, tbx2, tby2 = rows
    px1 = pred[0:1, :]
    py1 = pred[1:2, :]
    px2 = pred[2:3, :]
    py2 = pred[3:4, :]
    wp = jnp.maximum(jnp.minimum(tbx2, px2) - jnp.maximum(tbx1, px1), 0.0)
    hp = jnp.maximum(jnp.minimum(tby2, py2) - jnp.maximum(tby1, py1), 0.0)
    interp = wp * hp
    a_tb = (jnp.maximum(tbx2 - tbx1, 0.0) *
            jnp.maximum(tby2 - tby1, 0.0))
    a_pr = jnp.maximum(px2 - px1, 0.0) * jnp.maximum(py2 - py1, 0.0)
    iou_pred = interp / (a_tb + a_pr - interp + EPS)
    ioum = jnp.where(fg > 0.0, iou_pred, 0.0)
    cls_iota = jax.lax.broadcasted_iota(jnp.int32, (NUM_CLASSES, m), 0)
    sc_out[0] = jnp.where(cls_iota == labels, ioum, 0.0)


def _run(levels, anchors_t, gt_bboxes, labels_i, pred_t, interpret=False):
    b, n = gt_bboxes.shape[0], gt_bboxes.shape[1]
    m = anchors_t.shape[1]
    body = functools.partial(_atss_body, levels, n, m)
    out_shapes = (
        jax.ShapeDtypeStruct((b, 1, m), jnp.int32),
        jax.ShapeDtypeStruct((b, 4, m), jnp.float32),
        jax.ShapeDtypeStruct((b, NUM_CLASSES, m), jnp.float32),
        jax.ShapeDtypeStruct((b, 1, m), jnp.int32),
    )
    return pl.pallas_call(
        body,
        grid=(b,),
        in_specs=[
            pl.BlockSpec((4, m), lambda i: (0, 0)),
            pl.BlockSpec((1, n, 4), lambda i: (i, 0, 0)),
            pl.BlockSpec((1, n, 1), lambda i: (i, 0, 0)),
            pl.BlockSpec((1, 4, m), lambda i: (i, 0, 0)),
        ],
        out_specs=(
            pl.BlockSpec((1, 1, m), lambda i: (i, 0, 0)),
            pl.BlockSpec((1, 4, m), lambda i: (i, 0, 0)),
            pl.BlockSpec((1, NUM_CLASSES, m), lambda i: (i, 0, 0)),
            pl.BlockSpec((1, 1, m), lambda i: (i, 0, 0)),
        ),
        out_shape=out_shapes,
        compiler_params=pltpu.CompilerParams(
            dimension_semantics=("parallel",)),
        interpret=interpret,
    )(anchors_t, gt_bboxes, labels_i, pred_t)


def kernel(anchor_bboxes, n_level_bboxes, gt_bboxes, gt_labels, mask_gt,
           pred_bboxes, interpret=False):
    n_lv = len(n_level_bboxes)
    m = anchor_bboxes.shape[0]
    base = m // sum(4 ** j for j in range(n_lv))
    levels = tuple(base * (4 ** (n_lv - 1 - i)) for i in range(n_lv))

    anchors_t = anchor_bboxes.T.astype(jnp.float32)
    pred_t = jnp.transpose(pred_bboxes, (0, 2, 1)).astype(jnp.float32)
    labels_i = gt_labels.astype(jnp.int32)
    gt_f = gt_bboxes.astype(jnp.float32)

    labels, box_t, sc_t, fg = _run(levels, anchors_t, gt_f, labels_i,
                                   pred_t, interpret=interpret)
    target_bboxes = jnp.transpose(box_t, (0, 2, 1))
    target_scores = jnp.transpose(sc_t, (0, 2, 1))
    return labels[:, 0, :], target_bboxes, target_scores, fg[:, 0, :] > 0
